# baseline (speedup 1.0000x reference)
import jax
import jax.numpy as jnp
from jax import lax
from jax.experimental import pallas as pl
from jax.experimental.pallas import tpu as pltpu
from jax.experimental.pallas import tpu_sc as plsc

B = 16384
ED = 64
RD = 32
NW = 32
PASS = 128
NPASS = B // (NW * PASS)
CH = 8
NCH = PASS // CH


def _sc_body(head_hbm, rel_hbm, tail_hbm, ent2_hbm, relp_hbm, proj_hbm,
             out_hbm, hraw, traw, rebuf, obuf, pb0, pb1,
             hidx, tidx, ridx, pidx_h, pidx_t, hbit_h, hbit_t,
             sem_g, sem_p0, sem_p1):
  wid = lax.axis_index("s") * 2 + lax.axis_index("c")
  iota = lax.iota(jnp.int32, 16)
  ji = iota * ED

  def one_pass(p, carry):
    brow = wid * NPASS + p
    base = brow * PASS

    pltpu.sync_copy(head_hbm.at[pl.ds(brow, 1)], hidx)
    pltpu.sync_copy(tail_hbm.at[pl.ds(brow, 1)], tidx)
    pltpu.sync_copy(rel_hbm.at[pl.ds(brow, 1)], ridx)

    for c in range(PASS // 16):
      hv = hidx[0, pl.ds(c * 16, 16)]
      tv = tidx[0, pl.ds(c * 16, 16)]
      pidx_h[0, pl.ds(c * 16, 16)] = hv >> 1
      pidx_t[0, pl.ds(c * 16, 16)] = tv >> 1
      hbit_h[0, pl.ds(c * 16, 16)] = hv & 1
      hbit_t[0, pl.ds(c * 16, 16)] = tv & 1

    g1 = pltpu.make_async_copy(ent2_hbm.at[pidx_h.at[0]], hraw, sem_g)
    g2 = pltpu.make_async_copy(ent2_hbm.at[pidx_t.at[0]], traw, sem_g)
    g3 = pltpu.make_async_copy(relp_hbm.at[ridx.at[0]], rebuf, sem_g)
    g1.start(); g2.start(); g3.start()

    def p_desc(c, buf, sem):
      return pltpu.make_async_copy(
          proj_hbm.at[ridx.at[0, pl.ds(c * CH, CH)]], buf, sem)

    p_desc(0, pb0, sem_p0).start()
    p_desc(1, pb1, sem_p1).start()

    g1.wait(); g2.wait(); g3.wait()

    def dsub(g, carry2):
      hbv = hbit_h[0, pl.ds(g * 16, 16)]
      tbv = hbit_t[0, pl.ds(g * 16, 16)]
      for j in range(16):
        b = g * 16 + j
        bvec = jnp.broadcast_to(b, (16,))
        hoff = jnp.broadcast_to(hbv[j] * ED, (16,))
        toff = jnp.broadcast_to(tbv[j] * ED, (16,))
        for kk in range(ED // 16):
          col = iota + kk * 16
          hval = plsc.load_gather(hraw, [bvec, hoff + col])
          tval = plsc.load_gather(traw, [bvec, toff + col])
          traw[b, pl.ds(kk * 16, 16)] = hval - tval
      return carry2
    lax.fori_loop(0, PASS // 16, dsub, 0)

    def ring(it, carry2):
      for bb, (buf, sem) in enumerate(((pb0, sem_p0), (pb1, sem_p1))):
        c = it * 2 + bb
        p_desc(c, buf, sem).wait()

        def triple(s, carry3):
          b = c * CH + s
          svec = jnp.broadcast_to(s, (16,))
          a0 = rebuf[b, pl.ds(0, 16)]
          a1 = rebuf[b, pl.ds(16, 16)]
          for kk in range(ED // 16):
            dv = traw[b, pl.ds(kk * 16, 16)]
            for j in range(16):
              k = kk * 16 + j
              bc = jnp.broadcast_to(dv[j], (16,))
              p0 = plsc.load_gather(buf, [svec, ji + k])
              p1 = plsc.load_gather(buf, [svec, ji + (16 * ED + k)])
              a0 = a0 + bc * p0
              a1 = a1 + bc * p1
          obuf[b, pl.ds(0, 16)] = a0
          obuf[b, pl.ds(16, 16)] = a1
          return carry3
        lax.fori_loop(0, CH, triple, 0)

        nxt = c + 2

        @pl.when(nxt < NCH)
        def _():
          p_desc(nxt, buf, sem).start()
      return carry2
    lax.fori_loop(0, NCH // 2, ring, 0)

    pltpu.sync_copy(obuf, out_hbm.at[pl.ds(base, PASS)])
    return carry
  lax.fori_loop(0, NPASS, one_pass, 0)


def _tc_norm_body(x_ref, o_ref):
  x = x_ref[...]
  o_ref[...] = jnp.sqrt(jnp.sum(x * x, axis=1))


def kernel(head, relation, tail, entity_table, relation_table, proj_table):
  head2 = head.reshape(128, 128).astype(jnp.int32)
  rel2 = relation.reshape(128, 128).astype(jnp.int32)
  tail2 = tail.reshape(128, 128).astype(jnp.int32)
  ent2 = entity_table.reshape(-1, 2 * ED)
  relp = jnp.pad(relation_table, ((0, 0), (0, 128 - RD)))

  sc = pl.kernel(
      _sc_body,
      out_type=jax.ShapeDtypeStruct((B, RD), jnp.float32),
      mesh=plsc.VectorSubcoreMesh(core_axis_name="c", subcore_axis_name="s"),
      compiler_params=pltpu.CompilerParams(use_tc_tiling_on_sc=True,
                                           needs_layout_passes=False),
      scratch_types=[
          pltpu.VMEM((PASS, 2 * ED), jnp.float32),
          pltpu.VMEM((PASS, 2 * ED), jnp.float32),
          pltpu.VMEM((PASS, 128), jnp.float32),
          pltpu.VMEM((PASS, RD), jnp.float32),
          pltpu.VMEM((CH, RD * ED), jnp.float32),
          pltpu.VMEM((CH, RD * ED), jnp.float32),
          pltpu.VMEM((1, PASS), jnp.int32),
          pltpu.VMEM((1, PASS), jnp.int32),
          pltpu.VMEM((1, PASS), jnp.int32),
          pltpu.VMEM((1, PASS), jnp.int32),
          pltpu.VMEM((1, PASS), jnp.int32),
          pltpu.VMEM((1, PASS), jnp.int32),
          pltpu.VMEM((1, PASS), jnp.int32),
          pltpu.SemaphoreType.DMA,
          pltpu.SemaphoreType.DMA,
          pltpu.SemaphoreType.DMA,
      ],
  )
  diff = sc(head2, rel2, tail2, ent2, relp, proj_table)

  out = pl.pallas_call(
      _tc_norm_body,
      grid=(16,),
      in_specs=[pl.BlockSpec((B // 16, RD), lambda i: (i, 0))],
      out_specs=pl.BlockSpec((B // 16,), lambda i: (i,)),
      out_shape=jax.ShapeDtypeStruct((B,), jnp.float32),
  )(diff)
  return out
